# baseline (speedup 1.0000x reference)
import functools

import jax
import jax.numpy as jnp
from jax.experimental import pallas as pl
from jax.experimental.pallas import tpu as pltpu

B = 4
S = 4096
D = 2048
E = 64
K = 8
SBLK = 512
NBLK = S // SBLK


def _router_body(vt_ref, text_ref, w1_ref, w2_ref, b_ref,
                 topi_ref, topv_ref, probs_ref, ent_ref, aux_ref,
                 acc_ref):
    i = pl.program_id(0)

    @pl.when(i == 0)
    def _init():
        acc_ref[...] = jnp.zeros_like(acc_ref)

    acc_ref[...] += jnp.sum(vt_ref[...], axis=1)

    @pl.when(i == NBLK - 1)
    def _finish():
        h = acc_ref[...] * (1.0 / S)
        logits = (jnp.dot(h, w1_ref[...], preferred_element_type=jnp.float32)
                  + jnp.dot(text_ref[...], w2_ref[...],
                            preferred_element_type=jnp.float32)
                  + b_ref[...])
        m = jnp.max(logits, axis=-1, keepdims=True)
        ex = jnp.exp(logits - m)
        probs = ex / jnp.sum(ex, axis=-1, keepdims=True)
        probs_ref[...] = probs

        ent = -jnp.sum(probs * jnp.log(probs + 1e-8)) * (1.0 / B)
        ent_ref[...] = ent.reshape(1, 1)
        mu = jnp.mean(probs, axis=0, keepdims=True)
        aux_ref[...] = jnp.mean((probs - mu) ** 2).reshape(1, 1)

        idxs = jax.lax.broadcasted_iota(jnp.int32, (B, E), 1)
        work = probs
        vals = []
        args = []
        for _ in range(K):
            cur = jnp.max(work, axis=-1, keepdims=True)
            is_max = work == cur
            arg = jnp.min(jnp.where(is_max, idxs, E), axis=-1,
                          keepdims=True)
            vals.append(cur)
            args.append(arg)
            work = jnp.where(idxs == arg, -jnp.inf, work)
        topv = jnp.concatenate(vals, axis=1)
        topi = jnp.concatenate(args, axis=1)
        topi_ref[...] = topi
        topv_ref[...] = topv / (jnp.sum(topv, axis=-1, keepdims=True) + 1e-8)


@functools.partial(jax.jit, static_argnames=())
def kernel(video_tokens, text_state, W, b):
    w1 = W[:D]
    w2 = W[D:]
    b2 = b.reshape(1, E)
    grid = (NBLK,)
    topi, topv, probs, ent, aux = pl.pallas_call(
        _router_body,
        grid=grid,
        in_specs=[
            pl.BlockSpec((B, SBLK, D), lambda i: (0, i, 0)),
            pl.BlockSpec((B, D), lambda i: (0, 0)),
            pl.BlockSpec((D, E), lambda i: (0, 0)),
            pl.BlockSpec((D, E), lambda i: (0, 0)),
            pl.BlockSpec((1, E), lambda i: (0, 0)),
        ],
        out_specs=[
            pl.BlockSpec((B, K), lambda i: (0, 0)),
            pl.BlockSpec((B, K), lambda i: (0, 0)),
            pl.BlockSpec((B, E), lambda i: (0, 0)),
            pl.BlockSpec((1, 1), lambda i: (0, 0)),
            pl.BlockSpec((1, 1), lambda i: (0, 0)),
        ],
        out_shape=[
            jax.ShapeDtypeStruct((B, K), jnp.int32),
            jax.ShapeDtypeStruct((B, K), jnp.float32),
            jax.ShapeDtypeStruct((B, E), jnp.float32),
            jax.ShapeDtypeStruct((1, 1), jnp.float32),
            jax.ShapeDtypeStruct((1, 1), jnp.float32),
        ],
        scratch_shapes=[pltpu.VMEM((B, D), jnp.float32)],
    )(video_tokens, text_state, w1, w2, b2)
    return (topi, topv, probs, ent.reshape(()), aux.reshape(()))


# speedup vs baseline: 1.0129x; 1.0129x over previous
import functools

import jax
import jax.numpy as jnp
from jax.experimental import pallas as pl
from jax.experimental.pallas import tpu as pltpu

B = 4
S = 4096
D = 2048
E = 64
K = 8
SBLK = 256
NBLK = S // SBLK


def _router_body(vt_ref, text_ref, w1_ref, w2_ref, b_ref,
                 topi_ref, topv_ref, probs_ref, ent_ref, aux_ref,
                 acc_ref):
    i = pl.program_id(0)

    @pl.when(i == 0)
    def _init():
        acc_ref[...] = jnp.zeros_like(acc_ref)

    acc_ref[...] += jnp.sum(vt_ref[...], axis=1)

    @pl.when(i == NBLK - 1)
    def _finish():
        h = acc_ref[...] * (1.0 / S)
        logits = (jnp.dot(h, w1_ref[...], preferred_element_type=jnp.float32)
                  + jnp.dot(text_ref[...], w2_ref[...],
                            preferred_element_type=jnp.float32)
                  + b_ref[...])
        m = jnp.max(logits, axis=-1, keepdims=True)
        ex = jnp.exp(logits - m)
        probs = ex / jnp.sum(ex, axis=-1, keepdims=True)
        probs_ref[...] = probs

        ent = -jnp.sum(probs * jnp.log(probs + 1e-8)) * (1.0 / B)
        ent_ref[...] = ent.reshape(1, 1)
        mu = jnp.mean(probs, axis=0, keepdims=True)
        aux_ref[...] = jnp.mean((probs - mu) ** 2).reshape(1, 1)

        idxs = jax.lax.broadcasted_iota(jnp.int32, (B, E), 1)
        work = probs
        vals = []
        args = []
        for _ in range(K):
            cur = jnp.max(work, axis=-1, keepdims=True)
            is_max = work == cur
            arg = jnp.min(jnp.where(is_max, idxs, E), axis=-1,
                          keepdims=True)
            vals.append(cur)
            args.append(arg)
            work = jnp.where(idxs == arg, -jnp.inf, work)
        topv = jnp.concatenate(vals, axis=1)
        topi = jnp.concatenate(args, axis=1)
        topi_ref[...] = topi
        topv_ref[...] = topv / (jnp.sum(topv, axis=-1, keepdims=True) + 1e-8)


@functools.partial(jax.jit, static_argnames=())
def kernel(video_tokens, text_state, W, b):
    w1 = W[:D]
    w2 = W[D:]
    b2 = b.reshape(1, E)
    grid = (NBLK,)
    topi, topv, probs, ent, aux = pl.pallas_call(
        _router_body,
        grid=grid,
        in_specs=[
            pl.BlockSpec((B, SBLK, D), lambda i: (0, i, 0)),
            pl.BlockSpec((B, D), lambda i: (0, 0)),
            pl.BlockSpec((D, E), lambda i: (0, 0)),
            pl.BlockSpec((D, E), lambda i: (0, 0)),
            pl.BlockSpec((1, E), lambda i: (0, 0)),
        ],
        out_specs=[
            pl.BlockSpec((B, K), lambda i: (0, 0)),
            pl.BlockSpec((B, K), lambda i: (0, 0)),
            pl.BlockSpec((B, E), lambda i: (0, 0)),
            pl.BlockSpec((1, 1), lambda i: (0, 0)),
            pl.BlockSpec((1, 1), lambda i: (0, 0)),
        ],
        out_shape=[
            jax.ShapeDtypeStruct((B, K), jnp.int32),
            jax.ShapeDtypeStruct((B, K), jnp.float32),
            jax.ShapeDtypeStruct((B, E), jnp.float32),
            jax.ShapeDtypeStruct((1, 1), jnp.float32),
            jax.ShapeDtypeStruct((1, 1), jnp.float32),
        ],
        scratch_shapes=[pltpu.VMEM((B, D), jnp.float32)],
    )(video_tokens, text_state, w1, w2, b2)
    return (topi, topv, probs, ent.reshape(()), aux.reshape(()))
